# baseline (speedup 1.0000x reference)
import functools
import math

import numpy as np
import jax
import jax.numpy as jnp
from jax import lax
from jax.experimental import pallas as pl
from jax.experimental.pallas import tpu as pltpu
from jax.experimental.pallas import tpu_sc as plsc

_EMBED_DIM = 128
_SEQ_LEN = 200
_BATCH = 1024
_SCALE = math.sqrt(float(_EMBED_DIM))

_NUM_CORES = 2
_NUM_SUBCORES = 16
_NUM_WORKERS = _NUM_CORES * _NUM_SUBCORES
_ROWS_PER_WORKER = _BATCH // _NUM_WORKERS
_GATHER_SPLITS = ((0, 128), (128, 72))
_NBUF = 4
_PE_REPLICAS = 8
_LANES = 16
_VECS_PER_DIM = _EMBED_DIM // _LANES


def _positional_encoding_np(seq_len, d_model):
    pos = np.arange(seq_len, dtype=np.float32)[:, None]
    i = np.arange(0, d_model, 2, dtype=np.float32)
    div = np.exp(-np.log(10000.0) * i / float(d_model))
    pe = np.zeros((seq_len, d_model), dtype=np.float32)
    pe[:, 0::2] = np.sin(pos * div)
    pe[:, 1::2] = np.cos(pos * div)
    return pe


def _sc_body(table_hbm, idx_hbm, pediv_hbm, out_hbm,
             idx_v, rows_v, pe_sh, psems, gsems, wsems):
    sid = lax.axis_index("s")
    wid = sid * _NUM_CORES + lax.axis_index("c")
    row0 = wid * _ROWS_PER_WORKER
    pltpu.sync_copy(pediv_hbm, pe_sh.at[sid % _PE_REPLICAS])
    pltpu.sync_copy(idx_hbm.at[pl.ds(row0 * _SEQ_LEN,
                                     _ROWS_PER_WORKER * _SEQ_LEN)], idx_v)

    def start_prefill(k):
        buf = k % _NBUF
        return pltpu.async_copy(pe_sh.at[sid % _PE_REPLICAS], rows_v.at[buf],
                                psems.at[buf])

    def start_gathers(k):
        buf = k % _NBUF
        handles = []
        for off, n in _GATHER_SPLITS:
            handles.append(pltpu.async_copy(
                table_hbm.at[idx_v.at[pl.ds(k * _SEQ_LEN + off, n)]],
                rows_v.at[buf, pl.ds(off, n)],
                gsems.at[buf],
                add=True,
            ))
        return handles

    def scale(buf):
        def scale_row(i, carry):
            for j in range(_VECS_PER_DIM):
                sl = pl.ds(j * _LANES, _LANES)
                rows_v[buf, i, sl] = rows_v[buf, i, sl] * _SCALE
            return carry
        lax.fori_loop(0, _SEQ_LEN, scale_row, None, unroll=4)

    phandles = {0: start_prefill(0), 1: start_prefill(1)}
    phandles.pop(0).wait()
    ghandles = {0: start_gathers(0)}
    whandles = {}
    for k in range(_ROWS_PER_WORKER):
        if k + 1 < _ROWS_PER_WORKER:
            phandles.pop(k + 1).wait()
            ghandles[k + 1] = start_gathers(k + 1)
        if k + 2 < _ROWS_PER_WORKER:
            if k - 2 >= 0:
                whandles.pop(k - 2).wait()
            phandles[k + 2] = start_prefill(k + 2)
        for h in ghandles.pop(k):
            h.wait()
        scale(k % _NBUF)
        whandles[k] = pltpu.async_copy(
            rows_v.at[k % _NBUF],
            out_hbm.at[pl.ds((row0 + k) * _SEQ_LEN, _SEQ_LEN)],
            wsems.at[k % _NBUF],
        )
    for k in sorted(whandles):
        whandles.pop(k).wait()


@functools.partial(jax.jit, static_argnames=())
def _embed_lookup(table, idx_flat, pe_div):
    mesh = plsc.VectorSubcoreMesh(
        core_axis_name="c", subcore_axis_name="s",
        num_cores=_NUM_CORES, num_subcores=_NUM_SUBCORES,
    )
    fn = pl.kernel(
        _sc_body,
        out_type=jax.ShapeDtypeStruct((_BATCH * _SEQ_LEN, _EMBED_DIM),
                                      jnp.float32),
        mesh=mesh,
        scratch_types=[
            pltpu.VMEM((_ROWS_PER_WORKER * _SEQ_LEN,), jnp.int32),
            pltpu.VMEM((_NBUF, _SEQ_LEN, _EMBED_DIM), jnp.float32),
            pltpu.VMEM_SHARED((_PE_REPLICAS, _SEQ_LEN, _EMBED_DIM),
                              jnp.float32),
            pltpu.SemaphoreType.DMA((_NBUF,)),
            pltpu.SemaphoreType.DMA((_NBUF,)),
            pltpu.SemaphoreType.DMA((_NBUF,)),
        ],
    )
    return fn(table, idx_flat, pe_div)


def kernel(x, table):
    pe_div = jnp.asarray(
        _positional_encoding_np(_SEQ_LEN, _EMBED_DIM) / np.float32(_SCALE))
    idx_flat = x.reshape(-1).astype(jnp.int32)
    out = _embed_lookup(table, idx_flat, pe_div)
    return out.reshape(_BATCH, _SEQ_LEN, _EMBED_DIM)


# speedup vs baseline: 1.0081x; 1.0081x over previous
import functools
import math

import numpy as np
import jax
import jax.numpy as jnp
from jax import lax
from jax.experimental import pallas as pl
from jax.experimental.pallas import tpu as pltpu
from jax.experimental.pallas import tpu_sc as plsc

_EMBED_DIM = 128
_SEQ_LEN = 200
_BATCH = 1024
_SCALE = math.sqrt(float(_EMBED_DIM))

_NUM_CORES = 2
_NUM_SUBCORES = 16
_NUM_WORKERS = _NUM_CORES * _NUM_SUBCORES
_ROWS_PER_WORKER = _BATCH // _NUM_WORKERS
_GATHER_SPLITS = ((0, 128), (128, 72))
_NBUF = 4
_PE_REPLICAS = 1
_LANES = 16
_VECS_PER_DIM = _EMBED_DIM // _LANES


def _positional_encoding_np(seq_len, d_model):
    pos = np.arange(seq_len, dtype=np.float32)[:, None]
    i = np.arange(0, d_model, 2, dtype=np.float32)
    div = np.exp(-np.log(10000.0) * i / float(d_model))
    pe = np.zeros((seq_len, d_model), dtype=np.float32)
    pe[:, 0::2] = np.sin(pos * div)
    pe[:, 1::2] = np.cos(pos * div)
    return pe


def _sc_body(table_hbm, idx_hbm, pediv_hbm, out_hbm,
             idx_v, rows_v, pe_sh, psems, gsems, wsems):
    sid = lax.axis_index("s")
    wid = sid * _NUM_CORES + lax.axis_index("c")
    row0 = wid * _ROWS_PER_WORKER
    pltpu.sync_copy(pediv_hbm, pe_sh.at[sid % _PE_REPLICAS])
    pltpu.sync_copy(idx_hbm.at[pl.ds(row0 * _SEQ_LEN,
                                     _ROWS_PER_WORKER * _SEQ_LEN)], idx_v)

    def start_prefill(k):
        buf = k % _NBUF
        return pltpu.async_copy(pe_sh.at[sid % _PE_REPLICAS], rows_v.at[buf],
                                psems.at[buf])

    def start_gathers(k):
        buf = k % _NBUF
        handles = []
        for off, n in _GATHER_SPLITS:
            handles.append(pltpu.async_copy(
                table_hbm.at[idx_v.at[pl.ds(k * _SEQ_LEN + off, n)]],
                rows_v.at[buf, pl.ds(off, n)],
                gsems.at[buf],
                add=False,
            ))
        return handles

    def scale(buf):
        def scale_row(i, carry):
            for j in range(_VECS_PER_DIM):
                sl = pl.ds(j * _LANES, _LANES)
                rows_v[buf, i, sl] = rows_v[buf, i, sl] * _SCALE
            return carry
        lax.fori_loop(0, _SEQ_LEN, scale_row, None, unroll=4)

    phandles = {0: start_prefill(0), 1: start_prefill(1)}
    phandles.pop(0).wait()
    ghandles = {0: start_gathers(0)}
    whandles = {}
    for k in range(_ROWS_PER_WORKER):
        if k + 1 < _ROWS_PER_WORKER:
            phandles.pop(k + 1).wait()
            ghandles[k + 1] = start_gathers(k + 1)
        if k + 2 < _ROWS_PER_WORKER:
            if k - 2 >= 0:
                whandles.pop(k - 2).wait()
            phandles[k + 2] = start_prefill(k + 2)
        for h in ghandles.pop(k):
            h.wait()
        scale(k % _NBUF)
        whandles[k] = pltpu.async_copy(
            rows_v.at[k % _NBUF],
            out_hbm.at[pl.ds((row0 + k) * _SEQ_LEN, _SEQ_LEN)],
            wsems.at[k % _NBUF],
        )
    for k in sorted(whandles):
        whandles.pop(k).wait()


@functools.partial(jax.jit, static_argnames=())
def _embed_lookup(table, idx_flat, pe_div):
    mesh = plsc.VectorSubcoreMesh(
        core_axis_name="c", subcore_axis_name="s",
        num_cores=_NUM_CORES, num_subcores=_NUM_SUBCORES,
    )
    fn = pl.kernel(
        _sc_body,
        out_type=jax.ShapeDtypeStruct((_BATCH * _SEQ_LEN, _EMBED_DIM),
                                      jnp.float32),
        mesh=mesh,
        scratch_types=[
            pltpu.VMEM((_ROWS_PER_WORKER * _SEQ_LEN,), jnp.int32),
            pltpu.VMEM((_NBUF, _SEQ_LEN, _EMBED_DIM), jnp.float32),
            pltpu.VMEM_SHARED((_PE_REPLICAS, _SEQ_LEN, _EMBED_DIM),
                              jnp.float32),
            pltpu.SemaphoreType.DMA((_NBUF,)),
            pltpu.SemaphoreType.DMA((_NBUF,)),
            pltpu.SemaphoreType.DMA((_NBUF,)),
        ],
    )
    return fn(table, idx_flat, pe_div)


def kernel(x, table):
    pe_div = jnp.asarray(
        _positional_encoding_np(_SEQ_LEN, _EMBED_DIM) / np.float32(_SCALE))
    idx_flat = x.reshape(-1).astype(jnp.int32)
    out = _embed_lookup(table, idx_flat, pe_div)
    return out.reshape(_BATCH, _SEQ_LEN, _EMBED_DIM)


# speedup vs baseline: 1.0244x; 1.0162x over previous
import functools
import math

import numpy as np
import jax
import jax.numpy as jnp
from jax import lax
from jax.experimental import pallas as pl
from jax.experimental.pallas import tpu as pltpu
from jax.experimental.pallas import tpu_sc as plsc

_EMBED_DIM = 128
_SEQ_LEN = 200
_BATCH = 1024
_SCALE = math.sqrt(float(_EMBED_DIM))

_NUM_CORES = 2
_NUM_SUBCORES = 16
_NUM_WORKERS = _NUM_CORES * _NUM_SUBCORES
_ROWS_PER_WORKER = _BATCH // _NUM_WORKERS
_GATHER_SPLITS = ((0, 128), (128, 72))
_NBUF = 4
_PE_REPLICAS = 1
_LANES = 16
_VECS_PER_DIM = _EMBED_DIM // _LANES


def _positional_encoding_np(seq_len, d_model):
    pos = np.arange(seq_len, dtype=np.float32)[:, None]
    i = np.arange(0, d_model, 2, dtype=np.float32)
    div = np.exp(-np.log(10000.0) * i / float(d_model))
    pe = np.zeros((seq_len, d_model), dtype=np.float32)
    pe[:, 0::2] = np.sin(pos * div)
    pe[:, 1::2] = np.cos(pos * div)
    return pe


def _sc_body(table_hbm, idx_hbm, pediv_hbm, out_hbm,
             idx_v, rows_v, pe_sh, psems, gsems, wsems):
    sid = lax.axis_index("s")
    wid = sid * _NUM_CORES + lax.axis_index("c")
    row0 = wid * _ROWS_PER_WORKER
    pltpu.sync_copy(pediv_hbm, pe_sh.at[sid % _PE_REPLICAS])
    pltpu.sync_copy(idx_hbm.at[pl.ds(row0 * _SEQ_LEN,
                                     _ROWS_PER_WORKER * _SEQ_LEN)], idx_v)

    def start_prefill(k):
        buf = k % _NBUF
        return pltpu.async_copy(pe_sh.at[sid % _PE_REPLICAS], rows_v.at[buf],
                                psems.at[buf])

    def start_gathers(k):
        buf = k % _NBUF
        handles = []
        for off, n in _GATHER_SPLITS:
            handles.append(pltpu.async_copy(
                table_hbm.at[idx_v.at[pl.ds(k * _SEQ_LEN + off, n)]],
                rows_v.at[buf, pl.ds(off, n)],
                gsems.at[buf],
                add=False,
            ))
        return handles

    def scale(buf):
        def scale_row(i, carry):
            for j in range(_VECS_PER_DIM):
                sl = pl.ds(j * _LANES, _LANES)
                rows_v[buf, i, sl] = rows_v[buf, i, sl] * _SCALE
            return carry
        lax.fori_loop(0, _SEQ_LEN, scale_row, None, unroll=4)

    phandles = {}
    ghandles = {0: start_gathers(0)}
    whandles = {}
    for k in range(_ROWS_PER_WORKER):
        if k + 1 < _ROWS_PER_WORKER:
            if k - 2 >= 0:
                whandles.pop(k - 2).wait()
            ghandles[k + 1] = start_gathers(k + 1)
        for h in ghandles.pop(k):
            h.wait()
        scale(k % _NBUF)
        whandles[k] = pltpu.async_copy(
            rows_v.at[k % _NBUF],
            out_hbm.at[pl.ds((row0 + k) * _SEQ_LEN, _SEQ_LEN)],
            wsems.at[k % _NBUF],
        )
    for k in sorted(whandles):
        whandles.pop(k).wait()


@functools.partial(jax.jit, static_argnames=())
def _embed_lookup(table, idx_flat, pe_div):
    mesh = plsc.VectorSubcoreMesh(
        core_axis_name="c", subcore_axis_name="s",
        num_cores=_NUM_CORES, num_subcores=_NUM_SUBCORES,
    )
    fn = pl.kernel(
        _sc_body,
        out_type=jax.ShapeDtypeStruct((_BATCH * _SEQ_LEN, _EMBED_DIM),
                                      jnp.float32),
        mesh=mesh,
        scratch_types=[
            pltpu.VMEM((_ROWS_PER_WORKER * _SEQ_LEN,), jnp.int32),
            pltpu.VMEM((_NBUF, _SEQ_LEN, _EMBED_DIM), jnp.float32),
            pltpu.VMEM_SHARED((_PE_REPLICAS, _SEQ_LEN, _EMBED_DIM),
                              jnp.float32),
            pltpu.SemaphoreType.DMA((_NBUF,)),
            pltpu.SemaphoreType.DMA((_NBUF,)),
            pltpu.SemaphoreType.DMA((_NBUF,)),
        ],
    )
    return fn(table, idx_flat, pe_div)


def kernel(x, table):
    pe_div = jnp.asarray(
        _positional_encoding_np(_SEQ_LEN, _EMBED_DIM) / np.float32(_SCALE))
    idx_flat = x.reshape(-1).astype(jnp.int32)
    out = _embed_lookup(table, idx_flat, pe_div)
    return out.reshape(_BATCH, _SEQ_LEN, _EMBED_DIM)


# speedup vs baseline: 1.0372x; 1.0124x over previous
import functools
import math

import numpy as np
import jax
import jax.numpy as jnp
from jax import lax
from jax.experimental import pallas as pl
from jax.experimental.pallas import tpu as pltpu
from jax.experimental.pallas import tpu_sc as plsc

_EMBED_DIM = 128
_SEQ_LEN = 200
_BATCH = 1024
_SCALE = math.sqrt(float(_EMBED_DIM))

_NUM_CORES = 2
_NUM_SUBCORES = 16
_NUM_WORKERS = _NUM_CORES * _NUM_SUBCORES
_ROWS_PER_WORKER = _BATCH // _NUM_WORKERS
_GATHER_SPLITS = ((0, 128), (128, 72))
_NBUF = 4
_LANES = 16
_VECS_PER_DIM = _EMBED_DIM // _LANES


def _positional_encoding_np(seq_len, d_model):
    pos = np.arange(seq_len, dtype=np.float32)[:, None]
    i = np.arange(0, d_model, 2, dtype=np.float32)
    div = np.exp(-np.log(10000.0) * i / float(d_model))
    pe = np.zeros((seq_len, d_model), dtype=np.float32)
    pe[:, 0::2] = np.sin(pos * div)
    pe[:, 1::2] = np.cos(pos * div)
    return pe


def _sc_body(table_hbm, idx_hbm, pediv_hbm, out_hbm,
             idx_v, rows_v, pe_sh, psems, gsems, wsems):
    sid = lax.axis_index("s")
    wid = sid * _NUM_CORES + lax.axis_index("c")
    row0 = wid * _ROWS_PER_WORKER
    @pl.when(sid == 0)
    def _():
        pltpu.sync_copy(pediv_hbm, pe_sh)
    pltpu.sync_copy(idx_hbm.at[pl.ds(row0 * _SEQ_LEN,
                                     _ROWS_PER_WORKER * _SEQ_LEN)], idx_v)
    plsc.subcore_barrier()

    def start_prefill(k):
        buf = k % _NBUF
        return pltpu.async_copy(pe_sh, rows_v.at[buf], psems.at[buf])

    def start_gathers(k):
        buf = k % _NBUF
        handles = []
        for off, n in _GATHER_SPLITS:
            handles.append(pltpu.async_copy(
                table_hbm.at[idx_v.at[pl.ds(k * _SEQ_LEN + off, n)]],
                rows_v.at[buf, pl.ds(off, n)],
                gsems.at[buf],
                add=True,
            ))
        return handles

    def scale(buf):
        def scale_row(i, carry):
            for j in range(_VECS_PER_DIM):
                sl = pl.ds(j * _LANES, _LANES)
                rows_v[buf, i, sl] = rows_v[buf, i, sl] * _SCALE
            return carry
        lax.fori_loop(0, _SEQ_LEN, scale_row, None, unroll=2)

    phandles = {0: start_prefill(0), 1: start_prefill(1)}
    phandles.pop(0).wait()
    ghandles = {0: start_gathers(0)}
    whandles = {}
    for k in range(_ROWS_PER_WORKER):
        if k + 1 < _ROWS_PER_WORKER:
            phandles.pop(k + 1).wait()
            ghandles[k + 1] = start_gathers(k + 1)
        if k + 2 < _ROWS_PER_WORKER:
            if k - 2 >= 0:
                whandles.pop(k - 2).wait()
            phandles[k + 2] = start_prefill(k + 2)
        for h in ghandles.pop(k):
            h.wait()
        scale(k % _NBUF)
        whandles[k] = pltpu.async_copy(
            rows_v.at[k % _NBUF],
            out_hbm.at[pl.ds((row0 + k) * _SEQ_LEN, _SEQ_LEN)],
            wsems.at[k % _NBUF],
        )
    for k in sorted(whandles):
        whandles.pop(k).wait()


@functools.partial(jax.jit, static_argnames=())
def _embed_lookup(table, idx_flat, pe_div):
    mesh = plsc.VectorSubcoreMesh(
        core_axis_name="c", subcore_axis_name="s",
        num_cores=_NUM_CORES, num_subcores=_NUM_SUBCORES,
    )
    fn = pl.kernel(
        _sc_body,
        out_type=jax.ShapeDtypeStruct((_BATCH * _SEQ_LEN, _EMBED_DIM),
                                      jnp.float32),
        mesh=mesh,
        scratch_types=[
            pltpu.VMEM((_ROWS_PER_WORKER * _SEQ_LEN,), jnp.int32),
            pltpu.VMEM((_NBUF, _SEQ_LEN, _EMBED_DIM), jnp.float32),
            pltpu.VMEM_SHARED((_SEQ_LEN, _EMBED_DIM), jnp.float32),
            pltpu.SemaphoreType.DMA((_NBUF,)),
            pltpu.SemaphoreType.DMA((_NBUF,)),
            pltpu.SemaphoreType.DMA((_NBUF,)),
        ],
    )
    return fn(table, idx_flat, pe_div)


def kernel(x, table):
    pe_div = jnp.asarray(
        _positional_encoding_np(_SEQ_LEN, _EMBED_DIM) / np.float32(_SCALE))
    idx_flat = x.reshape(-1).astype(jnp.int32)
    out = _embed_lookup(table, idx_flat, pe_div)
    return out.reshape(_BATCH, _SEQ_LEN, _EMBED_DIM)
